# Initial kernel scaffold; baseline (speedup 1.0000x reference)
import jax
import jax.numpy as jnp
from jax.experimental import pallas as pl


def kernel(x, edge_index, edge_feat, dist, W1, b1, W2, b2, W3, b3):
    raise NotImplementedError("write your pallas kernel here")


# speedup vs baseline: 1.5819x; 1.5819x over previous
import functools
import math

import jax
import jax.numpy as jnp
from jax import lax
from jax.experimental import pallas as pl
from jax.experimental.pallas import tpu as pltpu
from jax.experimental.pallas import tpu_sc as plsc

N_NODES = 10000
N_EDGES = 320000
DIM = 128
HALF = 64
ONSET = 0.8
CUT = 1.0
LN2 = math.log(2.0)

BE = 1280
NEB = N_EDGES // BE
BN = 400
NNB = N_NODES // BN
BF = 2000
NFB = N_NODES // BF
NSUB = 16
EPS = N_EDGES // NSUB
KCH = 80
NCH = EPS // KCH
NPAD = 10240
RPT = NPAD // NSUB
RZ = 128


def _softplus(v):
    return jnp.logaddexp(v, 0.0)


def _edge_stage_body(dist_ref, ef_ref, w1_ref, b1_ref, w2_ref, b2_ref,
                     g_ref, t_ref):
    d = dist_ref[0, 0, :]
    gamma = DIM / (CUT - 0.0)
    mu = (lax.broadcasted_iota(jnp.int32, (1, DIM), 1).astype(jnp.float32)
          * (CUT / (DIM - 1)))
    bf = jnp.exp(-gamma * (d[:, None] - mu) ** 2)
    h = _softplus(jnp.dot(bf, w1_ref[...], preferred_element_type=jnp.float32)
                  + b1_ref[0, :]) - LN2
    h = _softplus(jnp.dot(h, w2_ref[...], preferred_element_type=jnp.float32)
                  + b2_ref[0, :]) - LN2
    t = (d - ONSET) / (CUT - ONSET)
    ramp = 0.5 * (jnp.cos(jnp.pi * jnp.clip(t, 0.0, 1.0)) + 1.0)
    co = jnp.where(d < ONSET, 1.0, jnp.where(d > CUT, 0.0, ramp))
    g = ef_ref[...] * h * co[:, None]
    g_ref[0, 0] = jnp.log(jnp.abs(g[:, :HALF]))
    g_ref[1, 0] = jnp.log(jnp.abs(g[:, HALF:]))
    t_ref[0, 0] = jnp.where(g[:, :HALF] < 0.0, 1.0, 0.0)
    t_ref[1, 0] = jnp.where(g[:, HALF:] < 0.0, 1.0, 0.0)


def _node_stage_body(x_ref, a_ref, s_ref):
    xv = x_ref[...]
    a_ref[0, 0] = jnp.log(jnp.abs(xv[:, :HALF]))
    a_ref[1, 0] = jnp.log(jnp.abs(xv[:, HALF:]))
    s_ref[0, 0] = jnp.where(xv[:, :HALF] < 0.0, 1.0, 0.0)
    s_ref[1, 0] = jnp.where(xv[:, HALF:] < 0.0, 1.0, 0.0)


def _final_stage_body(hl_ref, hs_ref, w3_ref, b3_ref, out_ref):
    hl = hl_ref[...]
    hs = hs_ref[...]
    h = jnp.concatenate([hl[0], hl[1]], axis=1)
    sc = jnp.concatenate([hs[0], hs[1]], axis=1)
    parity = sc - 2.0 * jnp.floor(sc * 0.5)
    sign = 1.0 - 2.0 * parity
    hv = sign * jnp.exp(h)
    out_ref[...] = _softplus(
        jnp.dot(hv, w3_ref[...], preferred_element_type=jnp.float32)
        + b3_ref[0, :]) - LN2


def _sc_scatter_body(src2, dstv, a2, s2, g2, t2, zrows,
                     hlo_out, hsg_out,
                     hlog_sh, hsgn_sh, isrc, idst,
                     abuf, sbuf, gbuf, tbuf, obuf,
                     sem_a, sem_b, sem_c, sem_d):
    c = lax.axis_index("c")
    s = lax.axis_index("s")
    base = s * RPT

    pltpu.sync_copy(zrows, obuf)

    def zbody(j, carry):
        r0 = base + j * RZ
        pltpu.sync_copy(obuf, hlog_sh.at[pl.ds(r0, RZ)])
        pltpu.sync_copy(obuf, hsgn_sh.at[pl.ds(r0, RZ)])
        return carry

    lax.fori_loop(0, RPT // RZ, zbody, 0)
    plsc.subcore_barrier()

    e0 = s * EPS

    def ebody(i, carry):
        eoff = e0 + i * KCH
        goff = c * N_EDGES + eoff
        pltpu.sync_copy(src2.at[pl.ds(goff, KCH)], isrc)
        pltpu.sync_copy(dstv.at[pl.ds(eoff, KCH)], idst)
        cpa = pltpu.async_copy(a2.at[isrc], abuf, sem_a)
        cps = pltpu.async_copy(s2.at[isrc], sbuf, sem_b)
        cpg = pltpu.async_copy(g2.at[pl.ds(goff, KCH)], gbuf, sem_c)
        cpt = pltpu.async_copy(t2.at[pl.ds(goff, KCH)], tbuf, sem_d)
        cpa.wait()
        cps.wait()
        cpg.wait()
        cpt.wait()
        pltpu.sync_copy(abuf, hlog_sh.at[idst], add=True)
        pltpu.sync_copy(gbuf, hlog_sh.at[idst], add=True)
        pltpu.sync_copy(sbuf, hsgn_sh.at[idst], add=True)
        pltpu.sync_copy(tbuf, hsgn_sh.at[idst], add=True)
        return carry

    lax.fori_loop(0, NCH, ebody, 0)
    plsc.subcore_barrier()

    def obody(j, carry):
        r0 = base + j * RZ
        pltpu.sync_copy(hlog_sh.at[pl.ds(r0, RZ)], obuf)
        pltpu.sync_copy(obuf, hlo_out.at[pl.ds(c * NPAD + r0, RZ)])
        pltpu.sync_copy(hsgn_sh.at[pl.ds(r0, RZ)], obuf)
        pltpu.sync_copy(obuf, hsg_out.at[pl.ds(c * NPAD + r0, RZ)])
        return carry

    lax.fori_loop(0, RPT // RZ, obody, 0)


_edge_stage = pl.pallas_call(
    _edge_stage_body,
    grid=(NEB,),
    in_specs=[
        pl.BlockSpec((1, 1, BE), lambda i: (i, 0, 0)),
        pl.BlockSpec((BE, DIM), lambda i: (i, 0)),
        pl.BlockSpec((DIM, DIM), lambda i: (0, 0)),
        pl.BlockSpec((1, DIM), lambda i: (0, 0)),
        pl.BlockSpec((DIM, DIM), lambda i: (0, 0)),
        pl.BlockSpec((1, DIM), lambda i: (0, 0)),
    ],
    out_specs=[
        pl.BlockSpec((2, 1, BE, HALF), lambda i: (0, i, 0, 0)),
        pl.BlockSpec((2, 1, BE, HALF), lambda i: (0, i, 0, 0)),
    ],
    out_shape=[
        jax.ShapeDtypeStruct((2, NEB, BE, HALF), jnp.float32),
        jax.ShapeDtypeStruct((2, NEB, BE, HALF), jnp.float32),
    ],
)

_node_stage = pl.pallas_call(
    _node_stage_body,
    grid=(NNB,),
    in_specs=[pl.BlockSpec((BN, DIM), lambda i: (i, 0))],
    out_specs=[
        pl.BlockSpec((2, 1, BN, HALF), lambda i: (0, i, 0, 0)),
        pl.BlockSpec((2, 1, BN, HALF), lambda i: (0, i, 0, 0)),
    ],
    out_shape=[
        jax.ShapeDtypeStruct((2, NNB, BN, HALF), jnp.float32),
        jax.ShapeDtypeStruct((2, NNB, BN, HALF), jnp.float32),
    ],
)

_final_stage = pl.pallas_call(
    _final_stage_body,
    grid=(NFB,),
    in_specs=[
        pl.BlockSpec((2, BF, HALF), lambda i: (0, i, 0)),
        pl.BlockSpec((2, BF, HALF), lambda i: (0, i, 0)),
        pl.BlockSpec((DIM, DIM), lambda i: (0, 0)),
        pl.BlockSpec((1, DIM), lambda i: (0, 0)),
    ],
    out_specs=pl.BlockSpec((BF, DIM), lambda i: (i, 0)),
    out_shape=jax.ShapeDtypeStruct((N_NODES, DIM), jnp.float32),
)

_sc_scatter = functools.partial(
    pl.kernel,
    mesh=plsc.VectorSubcoreMesh(core_axis_name="c", subcore_axis_name="s"),
    compiler_params=pltpu.CompilerParams(use_tc_tiling_on_sc=False),
    out_type=[
        jax.ShapeDtypeStruct((2 * NPAD, HALF), jnp.float32),
        jax.ShapeDtypeStruct((2 * NPAD, HALF), jnp.float32),
    ],
    scratch_types=[
        pltpu.VMEM_SHARED((NPAD, HALF), jnp.float32),
        pltpu.VMEM_SHARED((NPAD, HALF), jnp.float32),
        pltpu.VMEM((KCH,), jnp.int32),
        pltpu.VMEM((KCH,), jnp.int32),
        pltpu.VMEM((KCH, HALF), jnp.float32),
        pltpu.VMEM((KCH, HALF), jnp.float32),
        pltpu.VMEM((KCH, HALF), jnp.float32),
        pltpu.VMEM((KCH, HALF), jnp.float32),
        pltpu.VMEM((RZ, HALF), jnp.float32),
        pltpu.SemaphoreType.DMA,
        pltpu.SemaphoreType.DMA,
        pltpu.SemaphoreType.DMA,
        pltpu.SemaphoreType.DMA,
    ],
)(_sc_scatter_body)


def kernel(x, edge_index, edge_feat, dist, W1, b1, W2, b2, W3, b3):
    src = edge_index[0].astype(jnp.int32)
    dst = edge_index[1].astype(jnp.int32)
    src2 = jnp.concatenate([src, src + N_NODES])
    dist3 = dist.reshape(NEB, 1, BE)
    b1r = b1.reshape(1, DIM)
    b2r = b2.reshape(1, DIM)
    b3r = b3.reshape(1, DIM)

    g4, t4 = _edge_stage(dist3, edge_feat, W1, b1r, W2, b2r)
    a4, s4 = _node_stage(x)

    g2 = g4.reshape(2 * N_EDGES, HALF)
    t2 = t4.reshape(2 * N_EDGES, HALF)
    a2 = a4.reshape(2 * N_NODES, HALF)
    s2 = s4.reshape(2 * N_NODES, HALF)
    zrows = jnp.zeros((RZ, HALF), jnp.float32)

    hlo, hsg = _sc_scatter(src2, dst, a2, s2, g2, t2, zrows)

    out = _final_stage(hlo.reshape(2, NPAD, HALF)[:, :N_NODES, :],
                       hsg.reshape(2, NPAD, HALF)[:, :N_NODES, :], W3, b3r)
    return out


# speedup vs baseline: 2.5544x; 1.6148x over previous
import functools
import math

import jax
import jax.numpy as jnp
from jax import lax
from jax.experimental import pallas as pl
from jax.experimental.pallas import tpu as pltpu
from jax.experimental.pallas import tpu_sc as plsc

N_NODES = 10000
N_EDGES = 320000
DIM = 128
HALF = 64
ONSET = 0.8
CUT = 1.0
LN2 = math.log(2.0)

BE = 1280
NEB = N_EDGES // BE
BN = 400
NNB = N_NODES // BN
BF = 2000
NFB = N_NODES // BF
NSUB = 16
EPS = N_EDGES // NSUB
KCH = 40
NCH = EPS // KCH
NPAIR = NCH // 2
NPAD = 10240
RPT = NPAD // NSUB
RZ = 64


def _softplus(v):
    return jnp.logaddexp(v, 0.0)


def _edge_stage_body(dist_ref, ef_ref, w1_ref, b1_ref, w2_ref, b2_ref,
                     g_ref):
    d = dist_ref[0, 0, :]
    gamma = DIM / (CUT - 0.0)
    mu = (lax.broadcasted_iota(jnp.int32, (1, DIM), 1).astype(jnp.float32)
          * (CUT / (DIM - 1)))
    bf = jnp.exp(-gamma * (d[:, None] - mu) ** 2)
    h = _softplus(jnp.dot(bf, w1_ref[...], preferred_element_type=jnp.float32)
                  + b1_ref[0, :]) - LN2
    h = _softplus(jnp.dot(h, w2_ref[...], preferred_element_type=jnp.float32)
                  + b2_ref[0, :]) - LN2
    t = (d - ONSET) / (CUT - ONSET)
    ramp = 0.5 * (jnp.cos(jnp.pi * jnp.clip(t, 0.0, 1.0)) + 1.0)
    co = jnp.where(d < ONSET, 1.0, jnp.where(d > CUT, 0.0, ramp))
    g = ef_ref[...] * h * co[:, None]
    gl = jnp.log(jnp.abs(g))
    gs = jnp.where(g < 0.0, 1.0, 0.0)
    g_ref[0, 0] = jnp.concatenate([gl[:, :HALF], gs[:, :HALF]], axis=1)
    g_ref[1, 0] = jnp.concatenate([gl[:, HALF:], gs[:, HALF:]], axis=1)


def _node_stage_body(x_ref, a_ref):
    xv = x_ref[...]
    al = jnp.log(jnp.abs(xv))
    asg = jnp.where(xv < 0.0, 1.0, 0.0)
    a_ref[0, 0] = jnp.concatenate([al[:, :HALF], asg[:, :HALF]], axis=1)
    a_ref[1, 0] = jnp.concatenate([al[:, HALF:], asg[:, HALF:]], axis=1)


def _final_stage_body(hc_ref, w3_ref, b3_ref, out_ref):
    hc = hc_ref[...]
    h = jnp.concatenate([hc[0, :, :HALF], hc[1, :, :HALF]], axis=1)
    sc = jnp.concatenate([hc[0, :, HALF:], hc[1, :, HALF:]], axis=1)
    parity = sc - 2.0 * jnp.floor(sc * 0.5)
    sign = 1.0 - 2.0 * parity
    hv = sign * jnp.exp(h)
    out_ref[...] = _softplus(
        jnp.dot(hv, w3_ref[...], preferred_element_type=jnp.float32)
        + b3_ref[0, :]) - LN2


def _sc_scatter_body(src2, dstv, a2, g2, zrows,
                     acc_out,
                     acc_sh, isrc_a, idst_a, isrc_b, idst_b,
                     abuf_a, gbuf_a, abuf_b, gbuf_b, obuf,
                     sem_aa, sem_ga, sem_ab, sem_gb):
    c = lax.axis_index("c")
    s = lax.axis_index("s")
    base = s * RPT

    pltpu.sync_copy(zrows, obuf)

    def zbody(j, carry):
        pltpu.sync_copy(obuf, acc_sh.at[pl.ds(base + j * RZ, RZ)])
        return carry

    lax.fori_loop(0, RPT // RZ, zbody, 0)
    plsc.subcore_barrier()

    e0 = s * EPS

    def load_idx(chunk, isrc, idst):
        eoff = e0 + chunk * KCH
        pltpu.sync_copy(src2.at[pl.ds(c * N_EDGES + eoff, KCH)], isrc)
        pltpu.sync_copy(dstv.at[pl.ds(eoff, KCH)], idst)

    def start_gathers(chunk, isrc, abuf, gbuf, sem_a, sem_g):
        eoff = e0 + chunk * KCH
        cpa = pltpu.async_copy(a2.at[isrc], abuf, sem_a)
        cpg = pltpu.async_copy(g2.at[pl.ds(c * N_EDGES + eoff, KCH)],
                               gbuf, sem_g)
        return cpa, cpg

    def wait_and_scatter(chunk, isrc, abuf, gbuf, sem_a, sem_g, idst):
        eoff = e0 + chunk * KCH
        pltpu.make_async_copy(a2.at[isrc], abuf, sem_a).wait()
        pltpu.make_async_copy(g2.at[pl.ds(c * N_EDGES + eoff, KCH)],
                              gbuf, sem_g).wait()
        pltpu.sync_copy(abuf, acc_sh.at[idst], add=True)
        pltpu.sync_copy(gbuf, acc_sh.at[idst], add=True)

    load_idx(0, isrc_a, idst_a)
    start_gathers(0, isrc_a, abuf_a, gbuf_a, sem_aa, sem_ga)

    def ebody(i, carry):
        ca = 2 * i
        cb = 2 * i + 1
        load_idx(cb, isrc_b, idst_b)
        start_gathers(cb, isrc_b, abuf_b, gbuf_b, sem_ab, sem_gb)
        wait_and_scatter(ca, isrc_a, abuf_a, gbuf_a, sem_aa, sem_ga, idst_a)

        @pl.when(i < NPAIR - 1)
        def _():
            load_idx(ca + 2, isrc_a, idst_a)
            start_gathers(ca + 2, isrc_a, abuf_a, gbuf_a, sem_aa, sem_ga)

        wait_and_scatter(cb, isrc_b, abuf_b, gbuf_b, sem_ab, sem_gb, idst_b)
        return carry

    lax.fori_loop(0, NPAIR, ebody, 0)
    plsc.subcore_barrier()

    def obody(j, carry):
        r0 = base + j * RZ
        pltpu.sync_copy(acc_sh.at[pl.ds(r0, RZ)], obuf)
        pltpu.sync_copy(obuf, acc_out.at[pl.ds(c * NPAD + r0, RZ)])
        return carry

    lax.fori_loop(0, RPT // RZ, obody, 0)


_edge_stage = pl.pallas_call(
    _edge_stage_body,
    grid=(NEB,),
    in_specs=[
        pl.BlockSpec((1, 1, BE), lambda i: (i, 0, 0)),
        pl.BlockSpec((BE, DIM), lambda i: (i, 0)),
        pl.BlockSpec((DIM, DIM), lambda i: (0, 0)),
        pl.BlockSpec((1, DIM), lambda i: (0, 0)),
        pl.BlockSpec((DIM, DIM), lambda i: (0, 0)),
        pl.BlockSpec((1, DIM), lambda i: (0, 0)),
    ],
    out_specs=pl.BlockSpec((2, 1, BE, DIM), lambda i: (0, i, 0, 0)),
    out_shape=jax.ShapeDtypeStruct((2, NEB, BE, DIM), jnp.float32),
)

_node_stage = pl.pallas_call(
    _node_stage_body,
    grid=(NNB,),
    in_specs=[pl.BlockSpec((BN, DIM), lambda i: (i, 0))],
    out_specs=pl.BlockSpec((2, 1, BN, DIM), lambda i: (0, i, 0, 0)),
    out_shape=jax.ShapeDtypeStruct((2, NNB, BN, DIM), jnp.float32),
)

_final_stage = pl.pallas_call(
    _final_stage_body,
    grid=(NFB,),
    in_specs=[
        pl.BlockSpec((2, BF, DIM), lambda i: (0, i, 0)),
        pl.BlockSpec((DIM, DIM), lambda i: (0, 0)),
        pl.BlockSpec((1, DIM), lambda i: (0, 0)),
    ],
    out_specs=pl.BlockSpec((BF, DIM), lambda i: (i, 0)),
    out_shape=jax.ShapeDtypeStruct((N_NODES, DIM), jnp.float32),
)

_sc_scatter = functools.partial(
    pl.kernel,
    mesh=plsc.VectorSubcoreMesh(core_axis_name="c", subcore_axis_name="s"),
    compiler_params=pltpu.CompilerParams(use_tc_tiling_on_sc=False),
    out_type=jax.ShapeDtypeStruct((2 * NPAD, DIM), jnp.float32),
    scratch_types=[
        pltpu.VMEM_SHARED((NPAD, DIM), jnp.float32),
        pltpu.VMEM((KCH,), jnp.int32),
        pltpu.VMEM((KCH,), jnp.int32),
        pltpu.VMEM((KCH,), jnp.int32),
        pltpu.VMEM((KCH,), jnp.int32),
        pltpu.VMEM((KCH, DIM), jnp.float32),
        pltpu.VMEM((KCH, DIM), jnp.float32),
        pltpu.VMEM((KCH, DIM), jnp.float32),
        pltpu.VMEM((KCH, DIM), jnp.float32),
        pltpu.VMEM((RZ, DIM), jnp.float32),
        pltpu.SemaphoreType.DMA,
        pltpu.SemaphoreType.DMA,
        pltpu.SemaphoreType.DMA,
        pltpu.SemaphoreType.DMA,
    ],
)(_sc_scatter_body)


def kernel(x, edge_index, edge_feat, dist, W1, b1, W2, b2, W3, b3):
    src = edge_index[0].astype(jnp.int32)
    dst = edge_index[1].astype(jnp.int32)
    src2 = jnp.concatenate([src, src + N_NODES])
    dist3 = dist.reshape(NEB, 1, BE)
    b1r = b1.reshape(1, DIM)
    b2r = b2.reshape(1, DIM)
    b3r = b3.reshape(1, DIM)

    g4 = _edge_stage(dist3, edge_feat, W1, b1r, W2, b2r)
    a4 = _node_stage(x)

    g2 = g4.reshape(2 * N_EDGES, DIM)
    a2 = a4.reshape(2 * N_NODES, DIM)
    zrows = jnp.zeros((RZ, DIM), jnp.float32)

    acc = _sc_scatter(src2, dst, a2, g2, zrows)

    out = _final_stage(acc.reshape(2, NPAD, DIM)[:, :N_NODES, :], W3, b3r)
    return out


# speedup vs baseline: 3.2099x; 1.2566x over previous
import functools
import math

import jax
import jax.numpy as jnp
from jax import lax
from jax.experimental import pallas as pl
from jax.experimental.pallas import tpu as pltpu
from jax.experimental.pallas import tpu_sc as plsc

N_NODES = 10000
N_EDGES = 320000
DIM = 128
HALF = 64
ONSET = 0.8
CUT = 1.0
LN2 = math.log(2.0)

BE = 1280
NEB = N_EDGES // BE
BN = 400
NNB = N_NODES // BN
BF = 2000
NFB = N_NODES // BF
NSUB = 16
EPS = N_EDGES // NSUB
KCH = 80
NCH = EPS // KCH
NPAIR = NCH // 2
NPAD = 10240
RPT = NPAD // NSUB
RZ = 64


def _softplus(v):
    return jnp.logaddexp(v, 0.0)


def _edge_stage_body(dist_ref, ef_ref, w1_ref, b1_ref, w2_ref, b2_ref,
                     g_ref):
    d = dist_ref[0, 0, :]
    gamma = DIM / (CUT - 0.0)
    mu = (lax.broadcasted_iota(jnp.int32, (1, DIM), 1).astype(jnp.float32)
          * (CUT / (DIM - 1)))
    bf = jnp.exp(-gamma * (d[:, None] - mu) ** 2)
    h = _softplus(jnp.dot(bf, w1_ref[...], preferred_element_type=jnp.float32)
                  + b1_ref[0, :]) - LN2
    h = _softplus(jnp.dot(h, w2_ref[...], preferred_element_type=jnp.float32)
                  + b2_ref[0, :]) - LN2
    t = (d - ONSET) / (CUT - ONSET)
    ramp = 0.5 * (jnp.cos(jnp.pi * jnp.clip(t, 0.0, 1.0)) + 1.0)
    co = jnp.where(d < ONSET, 1.0, jnp.where(d > CUT, 0.0, ramp))
    g = ef_ref[...] * h * co[:, None]
    gl = jnp.log(jnp.abs(g))
    gs = jnp.where(g < 0.0, 1.0, 0.0)
    g_ref[0, 0] = jnp.concatenate([gl[:, :HALF], gs[:, :HALF]], axis=1)
    g_ref[1, 0] = jnp.concatenate([gl[:, HALF:], gs[:, HALF:]], axis=1)


def _node_stage_body(x_ref, a_ref):
    xv = x_ref[...]
    al = jnp.log(jnp.abs(xv))
    asg = jnp.where(xv < 0.0, 1.0, 0.0)
    a_ref[0, 0] = jnp.concatenate([al[:, :HALF], asg[:, :HALF]], axis=1)
    a_ref[1, 0] = jnp.concatenate([al[:, HALF:], asg[:, HALF:]], axis=1)


def _final_stage_body(hc_ref, w3_ref, b3_ref, out_ref):
    hc = hc_ref[...]
    h = jnp.concatenate([hc[0, :, :HALF], hc[1, :, :HALF]], axis=1)
    sc = jnp.concatenate([hc[0, :, HALF:], hc[1, :, HALF:]], axis=1)
    parity = sc - 2.0 * jnp.floor(sc * 0.5)
    sign = 1.0 - 2.0 * parity
    hv = sign * jnp.exp(h)
    out_ref[...] = _softplus(
        jnp.dot(hv, w3_ref[...], preferred_element_type=jnp.float32)
        + b3_ref[0, :]) - LN2


def _sc_scatter_body(ipk, a2, g2, zrows,
                     acc_out,
                     acc_sh, ibuf_a, ibuf_b,
                     abuf_a, gbuf_a, abuf_b, gbuf_b,
                     sem_aa, sem_ga, sem_ab, sem_gb, sem_s1, sem_s2):
    c = lax.axis_index("c")
    s = lax.axis_index("s")
    base = s * RPT

    pltpu.sync_copy(zrows, abuf_a.at[pl.ds(0, RZ)])

    def zbody(j, carry):
        pltpu.sync_copy(abuf_a.at[pl.ds(0, RZ)],
                        acc_sh.at[pl.ds(base + j * RZ, RZ)])
        return carry

    lax.fori_loop(0, RPT // RZ, zbody, 0)
    plsc.subcore_barrier()

    e0 = s * EPS
    lin0 = (c * NSUB + s) * NCH

    def load_idx(chunk, ibuf):
        pltpu.sync_copy(ipk.at[lin0 + chunk], ibuf)

    def start_gathers(chunk, ibuf, abuf, gbuf, sem_a, sem_g):
        eoff = e0 + chunk * KCH
        pltpu.async_copy(a2.at[ibuf.at[0]], abuf, sem_a)
        pltpu.async_copy(g2.at[pl.ds(c * N_EDGES + eoff, KCH)],
                         gbuf, sem_g)

    def wait_and_scatter(chunk, ibuf, abuf, gbuf, sem_a, sem_g):
        eoff = e0 + chunk * KCH
        pltpu.make_async_copy(a2.at[ibuf.at[0]], abuf, sem_a).wait()
        pltpu.make_async_copy(g2.at[pl.ds(c * N_EDGES + eoff, KCH)],
                              gbuf, sem_g).wait()
        ca = pltpu.async_copy(abuf, acc_sh.at[ibuf.at[1]], sem_s1, add=True)
        cg = pltpu.async_copy(gbuf, acc_sh.at[ibuf.at[1]], sem_s2, add=True)
        ca.wait()
        cg.wait()

    load_idx(0, ibuf_a)
    start_gathers(0, ibuf_a, abuf_a, gbuf_a, sem_aa, sem_ga)

    def ebody(i, carry):
        ca = 2 * i
        cb = 2 * i + 1
        load_idx(cb, ibuf_b)
        start_gathers(cb, ibuf_b, abuf_b, gbuf_b, sem_ab, sem_gb)
        wait_and_scatter(ca, ibuf_a, abuf_a, gbuf_a, sem_aa, sem_ga)

        @pl.when(i < NPAIR - 1)
        def _():
            load_idx(ca + 2, ibuf_a)
            start_gathers(ca + 2, ibuf_a, abuf_a, gbuf_a, sem_aa, sem_ga)

        wait_and_scatter(cb, ibuf_b, abuf_b, gbuf_b, sem_ab, sem_gb)
        return carry

    lax.fori_loop(0, NPAIR, ebody, 0)
    plsc.subcore_barrier()

    def obody(j, carry):
        r0 = base + j * RZ
        pltpu.sync_copy(acc_sh.at[pl.ds(r0, RZ)], abuf_a.at[pl.ds(0, RZ)])
        pltpu.sync_copy(abuf_a.at[pl.ds(0, RZ)],
                        acc_out.at[pl.ds(c * NPAD + r0, RZ)])
        return carry

    lax.fori_loop(0, RPT // RZ, obody, 0)


_edge_stage = pl.pallas_call(
    _edge_stage_body,
    grid=(NEB,),
    in_specs=[
        pl.BlockSpec((1, 1, BE), lambda i: (i, 0, 0)),
        pl.BlockSpec((BE, DIM), lambda i: (i, 0)),
        pl.BlockSpec((DIM, DIM), lambda i: (0, 0)),
        pl.BlockSpec((1, DIM), lambda i: (0, 0)),
        pl.BlockSpec((DIM, DIM), lambda i: (0, 0)),
        pl.BlockSpec((1, DIM), lambda i: (0, 0)),
    ],
    out_specs=pl.BlockSpec((2, 1, BE, DIM), lambda i: (0, i, 0, 0)),
    out_shape=jax.ShapeDtypeStruct((2, NEB, BE, DIM), jnp.float32),
)

_node_stage = pl.pallas_call(
    _node_stage_body,
    grid=(NNB,),
    in_specs=[pl.BlockSpec((BN, DIM), lambda i: (i, 0))],
    out_specs=pl.BlockSpec((2, 1, BN, DIM), lambda i: (0, i, 0, 0)),
    out_shape=jax.ShapeDtypeStruct((2, NNB, BN, DIM), jnp.float32),
)

_final_stage = pl.pallas_call(
    _final_stage_body,
    grid=(NFB,),
    in_specs=[
        pl.BlockSpec((2, BF, DIM), lambda i: (0, i, 0)),
        pl.BlockSpec((DIM, DIM), lambda i: (0, 0)),
        pl.BlockSpec((1, DIM), lambda i: (0, 0)),
    ],
    out_specs=pl.BlockSpec((BF, DIM), lambda i: (i, 0)),
    out_shape=jax.ShapeDtypeStruct((N_NODES, DIM), jnp.float32),
)

_sc_scatter = functools.partial(
    pl.kernel,
    mesh=plsc.VectorSubcoreMesh(core_axis_name="c", subcore_axis_name="s"),
    compiler_params=pltpu.CompilerParams(use_tc_tiling_on_sc=False),
    out_type=jax.ShapeDtypeStruct((2 * NPAD, DIM), jnp.float32),
    scratch_types=[
        pltpu.VMEM_SHARED((NPAD, DIM), jnp.float32),
        pltpu.VMEM((2, KCH), jnp.int32),
        pltpu.VMEM((2, KCH), jnp.int32),
        pltpu.VMEM((KCH, DIM), jnp.float32),
        pltpu.VMEM((KCH, DIM), jnp.float32),
        pltpu.VMEM((KCH, DIM), jnp.float32),
        pltpu.VMEM((KCH, DIM), jnp.float32),
        pltpu.SemaphoreType.DMA,
        pltpu.SemaphoreType.DMA,
        pltpu.SemaphoreType.DMA,
        pltpu.SemaphoreType.DMA,
        pltpu.SemaphoreType.DMA,
        pltpu.SemaphoreType.DMA,
    ],
)(_sc_scatter_body)


def kernel(x, edge_index, edge_feat, dist, W1, b1, W2, b2, W3, b3):
    src = edge_index[0].astype(jnp.int32)
    dst = edge_index[1].astype(jnp.int32)
    srcr = src.reshape(NSUB * NCH, KCH)
    dstr = dst.reshape(NSUB * NCH, KCH)
    ipk = jnp.concatenate(
        [jnp.stack([srcr, dstr], axis=1),
         jnp.stack([srcr + N_NODES, dstr], axis=1)], axis=0)
    dist3 = dist.reshape(NEB, 1, BE)
    b1r = b1.reshape(1, DIM)
    b2r = b2.reshape(1, DIM)
    b3r = b3.reshape(1, DIM)

    g4 = _edge_stage(dist3, edge_feat, W1, b1r, W2, b2r)
    a4 = _node_stage(x)

    g2 = g4.reshape(2 * N_EDGES, DIM)
    a2 = a4.reshape(2 * N_NODES, DIM)
    zrows = jnp.zeros((RZ, DIM), jnp.float32)

    acc = _sc_scatter(ipk, a2, g2, zrows)

    out = _final_stage(acc.reshape(2, NPAD, DIM)[:, :N_NODES, :], W3, b3r)
    return out


# speedup vs baseline: 3.9866x; 1.2420x over previous
import functools
import math

import jax
import jax.numpy as jnp
from jax import lax
from jax.experimental import pallas as pl
from jax.experimental.pallas import tpu as pltpu
from jax.experimental.pallas import tpu_sc as plsc

N_NODES = 10000
N_EDGES = 320000
NHALF = 2
EH = N_EDGES // NHALF
DIM = 128
HALF = 64
ONSET = 0.8
CUT = 1.0
LN2 = math.log(2.0)

BE = 1280
NEB_H = EH // BE
BN = 400
NNB = N_NODES // BN
BF = 2000
NFB = N_NODES // BF
NSUB = 16
EPS = EH // NSUB
KCH = 80
NCH = EPS // KCH
NPAIR = (NCH - 1) // 2
NPAD = 10240
RPT = NPAD // NSUB
RZ = 64


def _softplus(v):
    return jnp.logaddexp(v, 0.0)


def _edge_stage_body(dist_ref, ef_ref, w1_ref, b1_ref, w2_ref, b2_ref,
                     g_ref):
    d = dist_ref[0, 0, :]
    gamma = DIM / (CUT - 0.0)
    mu = (lax.broadcasted_iota(jnp.int32, (1, DIM), 1).astype(jnp.float32)
          * (CUT / (DIM - 1)))
    bf = jnp.exp(-gamma * (d[:, None] - mu) ** 2)
    h = _softplus(jnp.dot(bf, w1_ref[...], preferred_element_type=jnp.float32)
                  + b1_ref[0, :]) - LN2
    h = _softplus(jnp.dot(h, w2_ref[...], preferred_element_type=jnp.float32)
                  + b2_ref[0, :]) - LN2
    t = (d - ONSET) / (CUT - ONSET)
    ramp = 0.5 * (jnp.cos(jnp.pi * jnp.clip(t, 0.0, 1.0)) + 1.0)
    co = jnp.where(d < ONSET, 1.0, jnp.where(d > CUT, 0.0, ramp))
    g = ef_ref[...] * h * co[:, None]
    gl = jnp.log(jnp.abs(g))
    gs = jnp.where(g < 0.0, 1.0, 0.0)
    g_ref[0, 0] = jnp.concatenate([gl[:, :HALF], gs[:, :HALF]], axis=1)
    g_ref[1, 0] = jnp.concatenate([gl[:, HALF:], gs[:, HALF:]], axis=1)


def _node_stage_body(x_ref, a_ref):
    xv = x_ref[...]
    al = jnp.log(jnp.abs(xv))
    asg = jnp.where(xv < 0.0, 1.0, 0.0)
    a_ref[0, 0] = jnp.concatenate([al[:, :HALF], asg[:, :HALF]], axis=1)
    a_ref[1, 0] = jnp.concatenate([al[:, HALF:], asg[:, HALF:]], axis=1)


def _final_stage_body(h0_ref, h1_ref, w3_ref, b3_ref, out_ref):
    hc = h0_ref[...] + h1_ref[...]
    h = jnp.concatenate([hc[0, :, :HALF], hc[1, :, :HALF]], axis=1)
    sc = jnp.concatenate([hc[0, :, HALF:], hc[1, :, HALF:]], axis=1)
    parity = sc - 2.0 * jnp.floor(sc * 0.5)
    sign = 1.0 - 2.0 * parity
    hv = sign * jnp.exp(h)
    out_ref[...] = _softplus(
        jnp.dot(hv, w3_ref[...], preferred_element_type=jnp.float32)
        + b3_ref[0, :]) - LN2


def _sc_scatter_body(ipk, a2, g2, zrows,
                     acc_out,
                     acc_sh, ibuf_a, ibuf_b,
                     abuf_a, gbuf_a, abuf_b, gbuf_b,
                     sem_aa, sem_ga, sem_ab, sem_gb, sem_s1, sem_s2):
    c = lax.axis_index("c")
    s = lax.axis_index("s")
    base = s * RPT

    pltpu.sync_copy(zrows, abuf_a.at[pl.ds(0, RZ)])

    def zbody(j, carry):
        pltpu.sync_copy(abuf_a.at[pl.ds(0, RZ)],
                        acc_sh.at[pl.ds(base + j * RZ, RZ)])
        return carry

    lax.fori_loop(0, RPT // RZ, zbody, 0)
    plsc.subcore_barrier()

    e0 = s * EPS
    lin0 = (c * NSUB + s) * NCH

    def load_idx(chunk, ibuf):
        pltpu.sync_copy(ipk.at[lin0 + chunk], ibuf)

    def start_gathers(chunk, ibuf, abuf, gbuf, sem_a, sem_g):
        eoff = e0 + chunk * KCH
        pltpu.async_copy(a2.at[ibuf.at[0]], abuf, sem_a)
        pltpu.async_copy(g2.at[pl.ds(c * EH + eoff, KCH)],
                         gbuf, sem_g)

    def wait_and_scatter(chunk, ibuf, abuf, gbuf, sem_a, sem_g):
        eoff = e0 + chunk * KCH
        pltpu.make_async_copy(a2.at[ibuf.at[0]], abuf, sem_a).wait()
        pltpu.make_async_copy(g2.at[pl.ds(c * EH + eoff, KCH)],
                              gbuf, sem_g).wait()
        ca = pltpu.async_copy(abuf, acc_sh.at[ibuf.at[1]], sem_s1, add=True)
        cg = pltpu.async_copy(gbuf, acc_sh.at[ibuf.at[1]], sem_s2, add=True)
        ca.wait()
        cg.wait()

    load_idx(0, ibuf_a)
    start_gathers(0, ibuf_a, abuf_a, gbuf_a, sem_aa, sem_ga)

    def ebody(i, carry):
        ca = 2 * i
        cb = 2 * i + 1
        load_idx(cb, ibuf_b)
        start_gathers(cb, ibuf_b, abuf_b, gbuf_b, sem_ab, sem_gb)
        wait_and_scatter(ca, ibuf_a, abuf_a, gbuf_a, sem_aa, sem_ga)
        load_idx(ca + 2, ibuf_a)
        start_gathers(ca + 2, ibuf_a, abuf_a, gbuf_a, sem_aa, sem_ga)
        wait_and_scatter(cb, ibuf_b, abuf_b, gbuf_b, sem_ab, sem_gb)
        return carry

    lax.fori_loop(0, NPAIR, ebody, 0)
    wait_and_scatter(NCH - 1, ibuf_a, abuf_a, gbuf_a, sem_aa, sem_ga)
    plsc.subcore_barrier()

    def obody(j, carry):
        r0 = base + j * RZ
        pltpu.sync_copy(acc_sh.at[pl.ds(r0, RZ)], abuf_a.at[pl.ds(0, RZ)])
        pltpu.sync_copy(abuf_a.at[pl.ds(0, RZ)],
                        acc_out.at[pl.ds(c * NPAD + r0, RZ)])
        return carry

    lax.fori_loop(0, RPT // RZ, obody, 0)


def _make_edge_stage(h):
    return pl.pallas_call(
        _edge_stage_body,
        grid=(NEB_H,),
        in_specs=[
            pl.BlockSpec((1, 1, BE), lambda i: (h * NEB_H + i, 0, 0)),
            pl.BlockSpec((BE, DIM), lambda i: (h * NEB_H + i, 0)),
            pl.BlockSpec((DIM, DIM), lambda i: (0, 0)),
            pl.BlockSpec((1, DIM), lambda i: (0, 0)),
            pl.BlockSpec((DIM, DIM), lambda i: (0, 0)),
            pl.BlockSpec((1, DIM), lambda i: (0, 0)),
        ],
        out_specs=pl.BlockSpec((2, 1, BE, DIM), lambda i: (0, i, 0, 0)),
        out_shape=jax.ShapeDtypeStruct((2, NEB_H, BE, DIM), jnp.float32),
    )


_edge_stage_0 = _make_edge_stage(0)
_edge_stage_1 = _make_edge_stage(1)

_node_stage = pl.pallas_call(
    _node_stage_body,
    grid=(NNB,),
    in_specs=[pl.BlockSpec((BN, DIM), lambda i: (i, 0))],
    out_specs=pl.BlockSpec((2, 1, BN, DIM), lambda i: (0, i, 0, 0)),
    out_shape=jax.ShapeDtypeStruct((2, NNB, BN, DIM), jnp.float32),
)

_final_stage = pl.pallas_call(
    _final_stage_body,
    grid=(NFB,),
    in_specs=[
        pl.BlockSpec((2, BF, DIM), lambda i: (0, i, 0)),
        pl.BlockSpec((2, BF, DIM), lambda i: (0, i, 0)),
        pl.BlockSpec((DIM, DIM), lambda i: (0, 0)),
        pl.BlockSpec((1, DIM), lambda i: (0, 0)),
    ],
    out_specs=pl.BlockSpec((BF, DIM), lambda i: (i, 0)),
    out_shape=jax.ShapeDtypeStruct((N_NODES, DIM), jnp.float32),
)

_sc_scatter = functools.partial(
    pl.kernel,
    mesh=plsc.VectorSubcoreMesh(core_axis_name="c", subcore_axis_name="s"),
    compiler_params=pltpu.CompilerParams(use_tc_tiling_on_sc=False),
    out_type=jax.ShapeDtypeStruct((2 * NPAD, DIM), jnp.float32),
    scratch_types=[
        pltpu.VMEM_SHARED((NPAD, DIM), jnp.float32),
        pltpu.VMEM((2, KCH), jnp.int32),
        pltpu.VMEM((2, KCH), jnp.int32),
        pltpu.VMEM((KCH, DIM), jnp.float32),
        pltpu.VMEM((KCH, DIM), jnp.float32),
        pltpu.VMEM((KCH, DIM), jnp.float32),
        pltpu.VMEM((KCH, DIM), jnp.float32),
        pltpu.SemaphoreType.DMA,
        pltpu.SemaphoreType.DMA,
        pltpu.SemaphoreType.DMA,
        pltpu.SemaphoreType.DMA,
        pltpu.SemaphoreType.DMA,
        pltpu.SemaphoreType.DMA,
    ],
)(_sc_scatter_body)


def _pack_idx(srch, dsth):
    srcr = srch.reshape(NSUB * NCH, KCH)
    dstr = dsth.reshape(NSUB * NCH, KCH)
    return jnp.concatenate(
        [jnp.stack([srcr, dstr], axis=1),
         jnp.stack([srcr + N_NODES, dstr], axis=1)], axis=0)


def kernel(x, edge_index, edge_feat, dist, W1, b1, W2, b2, W3, b3):
    src = edge_index[0].astype(jnp.int32)
    dst = edge_index[1].astype(jnp.int32)
    ipk0 = _pack_idx(src[:EH], dst[:EH])
    ipk1 = _pack_idx(src[EH:], dst[EH:])
    dist3 = dist.reshape(NHALF * NEB_H, 1, BE)
    b1r = b1.reshape(1, DIM)
    b2r = b2.reshape(1, DIM)
    b3r = b3.reshape(1, DIM)

    a4 = _node_stage(x)
    a2 = a4.reshape(2 * N_NODES, DIM)
    zrows = jnp.zeros((RZ, DIM), jnp.float32)

    g4_0 = _edge_stage_0(dist3, edge_feat, W1, b1r, W2, b2r)
    acc0 = _sc_scatter(ipk0, a2, g4_0.reshape(2 * EH, DIM), zrows)

    g4_1 = _edge_stage_1(dist3, edge_feat, W1, b1r, W2, b2r)
    acc1 = _sc_scatter(ipk1, a2, g4_1.reshape(2 * EH, DIM), zrows)

    out = _final_stage(acc0.reshape(2, NPAD, DIM)[:, :N_NODES, :],
                       acc1.reshape(2, NPAD, DIM)[:, :N_NODES, :],
                       W3, b3r)
    return out


# speedup vs baseline: 4.1137x; 1.0319x over previous
import functools
import math

import jax
import jax.numpy as jnp
from jax import lax
from jax.experimental import pallas as pl
from jax.experimental.pallas import tpu as pltpu
from jax.experimental.pallas import tpu_sc as plsc

N_NODES = 10000
N_EDGES = 320000
DIM = 128
HALF = 64
ONSET = 0.8
CUT = 1.0
LN2 = math.log(2.0)

BE = 1280
NEB = N_EDGES // BE
SPLITS = [(0, 20), (20, 76), (96, 76), (172, 78)]
BN = 400
NNB = N_NODES // BN
BF = 2000
NFB = N_NODES // BF
NSUB = 16
KCH = 80
NPAD = 10240
RPT = NPAD // NSUB
RZ = 64


def _softplus(v):
    return jnp.logaddexp(v, 0.0)


def _edge_stage_body(dist_ref, ef_ref, w1_ref, b1_ref, w2_ref, b2_ref,
                     g_ref):
    d = dist_ref[0, 0, :]
    gamma = DIM / (CUT - 0.0)
    mu = (lax.broadcasted_iota(jnp.int32, (1, DIM), 1).astype(jnp.float32)
          * (CUT / (DIM - 1)))
    bf = jnp.exp(-gamma * (d[:, None] - mu) ** 2)
    h = _softplus(jnp.dot(bf, w1_ref[...], preferred_element_type=jnp.float32)
                  + b1_ref[0, :]) - LN2
    h = _softplus(jnp.dot(h, w2_ref[...], preferred_element_type=jnp.float32)
                  + b2_ref[0, :]) - LN2
    t = (d - ONSET) / (CUT - ONSET)
    ramp = 0.5 * (jnp.cos(jnp.pi * jnp.clip(t, 0.0, 1.0)) + 1.0)
    co = jnp.where(d < ONSET, 1.0, jnp.where(d > CUT, 0.0, ramp))
    g = ef_ref[...] * h * co[:, None]
    gl = jnp.log(jnp.abs(g))
    gs = jnp.where(g < 0.0, 1.0, 0.0)
    g_ref[0, 0] = jnp.concatenate([gl[:, :HALF], gs[:, :HALF]], axis=1)
    g_ref[1, 0] = jnp.concatenate([gl[:, HALF:], gs[:, HALF:]], axis=1)


def _node_stage_body(x_ref, a_ref):
    xv = x_ref[...]
    al = jnp.log(jnp.abs(xv))
    asg = jnp.where(xv < 0.0, 1.0, 0.0)
    a_ref[0, 0] = jnp.concatenate([al[:, :HALF], asg[:, :HALF]], axis=1)
    a_ref[1, 0] = jnp.concatenate([al[:, HALF:], asg[:, HALF:]], axis=1)


def _final_stage_body(h0_ref, h1_ref, h2_ref, h3_ref, w3_ref, b3_ref,
                      out_ref):
    hc = h0_ref[...] + h1_ref[...] + h2_ref[...] + h3_ref[...]
    h = jnp.concatenate([hc[0, :, :HALF], hc[1, :, :HALF]], axis=1)
    sc = jnp.concatenate([hc[0, :, HALF:], hc[1, :, HALF:]], axis=1)
    parity = sc - 2.0 * jnp.floor(sc * 0.5)
    sign = 1.0 - 2.0 * parity
    hv = sign * jnp.exp(h)
    out_ref[...] = _softplus(
        jnp.dot(hv, w3_ref[...], preferred_element_type=jnp.float32)
        + b3_ref[0, :]) - LN2


def _make_sc_scatter_body(nch):
    eps = nch * KCH
    esz = NSUB * eps
    npair = nch // 2

    def body(ipk, a2, g2, zrows,
             acc_out,
             acc_sh, ibuf_a, ibuf_b,
             abuf_a, gbuf_a, abuf_b, gbuf_b,
             sem_aa, sem_ga, sem_ab, sem_gb, sem_s1, sem_s2):
        c = lax.axis_index("c")
        s = lax.axis_index("s")
        base = s * RPT

        pltpu.sync_copy(zrows, abuf_a.at[pl.ds(0, RZ)])

        def zbody(j, carry):
            pltpu.sync_copy(abuf_a.at[pl.ds(0, RZ)],
                            acc_sh.at[pl.ds(base + j * RZ, RZ)])
            return carry

        lax.fori_loop(0, RPT // RZ, zbody, 0)
        plsc.subcore_barrier()

        e0 = s * eps
        lin0 = (c * NSUB + s) * nch

        def load_idx(chunk, ibuf):
            pltpu.sync_copy(ipk.at[lin0 + chunk], ibuf)

        def start_gathers(chunk, ibuf, abuf, gbuf, sem_a, sem_g):
            eoff = e0 + chunk * KCH
            pltpu.async_copy(a2.at[ibuf.at[0]], abuf, sem_a)
            pltpu.async_copy(g2.at[pl.ds(c * esz + eoff, KCH)],
                             gbuf, sem_g)

        def wait_and_scatter(chunk, ibuf, abuf, gbuf, sem_a, sem_g):
            eoff = e0 + chunk * KCH
            pltpu.make_async_copy(a2.at[ibuf.at[0]], abuf, sem_a).wait()
            pltpu.make_async_copy(g2.at[pl.ds(c * esz + eoff, KCH)],
                                  gbuf, sem_g).wait()
            ca = pltpu.async_copy(abuf, acc_sh.at[ibuf.at[1]], sem_s1,
                                  add=True)
            cg = pltpu.async_copy(gbuf, acc_sh.at[ibuf.at[1]], sem_s2,
                                  add=True)
            ca.wait()
            cg.wait()

        load_idx(0, ibuf_a)
        start_gathers(0, ibuf_a, abuf_a, gbuf_a, sem_aa, sem_ga)

        def ebody(i, carry):
            ca = 2 * i
            cb = 2 * i + 1
            load_idx(cb, ibuf_b)
            start_gathers(cb, ibuf_b, abuf_b, gbuf_b, sem_ab, sem_gb)
            wait_and_scatter(ca, ibuf_a, abuf_a, gbuf_a, sem_aa, sem_ga)

            @pl.when(i < npair - 1)
            def _():
                load_idx(ca + 2, ibuf_a)
                start_gathers(ca + 2, ibuf_a, abuf_a, gbuf_a, sem_aa,
                              sem_ga)

            wait_and_scatter(cb, ibuf_b, abuf_b, gbuf_b, sem_ab, sem_gb)
            return carry

        lax.fori_loop(0, npair, ebody, 0)
        plsc.subcore_barrier()

        def obody(j, carry):
            r0 = base + j * RZ
            pltpu.sync_copy(acc_sh.at[pl.ds(r0, RZ)],
                            abuf_a.at[pl.ds(0, RZ)])
            pltpu.sync_copy(abuf_a.at[pl.ds(0, RZ)],
                            acc_out.at[pl.ds(c * NPAD + r0, RZ)])
            return carry

        lax.fori_loop(0, RPT // RZ, obody, 0)

    return body


def _make_edge_stage(blk0, nb):
    return pl.pallas_call(
        _edge_stage_body,
        grid=(nb,),
        in_specs=[
            pl.BlockSpec((1, 1, BE), lambda i: (blk0 + i, 0, 0)),
            pl.BlockSpec((BE, DIM), lambda i: (blk0 + i, 0)),
            pl.BlockSpec((DIM, DIM), lambda i: (0, 0)),
            pl.BlockSpec((1, DIM), lambda i: (0, 0)),
            pl.BlockSpec((DIM, DIM), lambda i: (0, 0)),
            pl.BlockSpec((1, DIM), lambda i: (0, 0)),
        ],
        out_specs=pl.BlockSpec((2, 1, BE, DIM), lambda i: (0, i, 0, 0)),
        out_shape=jax.ShapeDtypeStruct((2, nb, BE, DIM), jnp.float32),
    )


def _make_sc_scatter(nch):
    return functools.partial(
        pl.kernel,
        mesh=plsc.VectorSubcoreMesh(core_axis_name="c", subcore_axis_name="s"),
        compiler_params=pltpu.CompilerParams(use_tc_tiling_on_sc=False),
        out_type=jax.ShapeDtypeStruct((2 * NPAD, DIM), jnp.float32),
        scratch_types=[
            pltpu.VMEM_SHARED((NPAD, DIM), jnp.float32),
            pltpu.VMEM((2, KCH), jnp.int32),
            pltpu.VMEM((2, KCH), jnp.int32),
            pltpu.VMEM((KCH, DIM), jnp.float32),
            pltpu.VMEM((KCH, DIM), jnp.float32),
            pltpu.VMEM((KCH, DIM), jnp.float32),
            pltpu.VMEM((KCH, DIM), jnp.float32),
            pltpu.SemaphoreType.DMA,
            pltpu.SemaphoreType.DMA,
            pltpu.SemaphoreType.DMA,
            pltpu.SemaphoreType.DMA,
            pltpu.SemaphoreType.DMA,
            pltpu.SemaphoreType.DMA,
        ],
    )(_make_sc_scatter_body(nch))


_edge_stages = [_make_edge_stage(b0, nb) for b0, nb in SPLITS]
_sc_scatters = [_make_sc_scatter(nb * BE // (NSUB * KCH)) for _, nb in SPLITS]

_node_stage = pl.pallas_call(
    _node_stage_body,
    grid=(NNB,),
    in_specs=[pl.BlockSpec((BN, DIM), lambda i: (i, 0))],
    out_specs=pl.BlockSpec((2, 1, BN, DIM), lambda i: (0, i, 0, 0)),
    out_shape=jax.ShapeDtypeStruct((2, NNB, BN, DIM), jnp.float32),
)

_final_stage = pl.pallas_call(
    _final_stage_body,
    grid=(NFB,),
    in_specs=[
        pl.BlockSpec((2, BF, DIM), lambda i: (0, i, 0)),
        pl.BlockSpec((2, BF, DIM), lambda i: (0, i, 0)),
        pl.BlockSpec((2, BF, DIM), lambda i: (0, i, 0)),
        pl.BlockSpec((2, BF, DIM), lambda i: (0, i, 0)),
        pl.BlockSpec((DIM, DIM), lambda i: (0, 0)),
        pl.BlockSpec((1, DIM), lambda i: (0, 0)),
    ],
    out_specs=pl.BlockSpec((BF, DIM), lambda i: (i, 0)),
    out_shape=jax.ShapeDtypeStruct((N_NODES, DIM), jnp.float32),
)


def _pack_idx(srch, dsth, nch):
    srcr = srch.reshape(NSUB * nch, KCH)
    dstr = dsth.reshape(NSUB * nch, KCH)
    return jnp.concatenate(
        [jnp.stack([srcr, dstr], axis=1),
         jnp.stack([srcr + N_NODES, dstr], axis=1)], axis=0)


def kernel(x, edge_index, edge_feat, dist, W1, b1, W2, b2, W3, b3):
    src = edge_index[0].astype(jnp.int32)
    dst = edge_index[1].astype(jnp.int32)
    dist3 = dist.reshape(NEB, 1, BE)
    b1r = b1.reshape(1, DIM)
    b2r = b2.reshape(1, DIM)
    b3r = b3.reshape(1, DIM)

    a4 = _node_stage(x)
    a2 = a4.reshape(2 * N_NODES, DIM)
    zrows = jnp.zeros((RZ, DIM), jnp.float32)

    accs = []
    for k, (b0, nb) in enumerate(SPLITS):
        e0 = b0 * BE
        esz = nb * BE
        nch = esz // (NSUB * KCH)
        ipk = _pack_idx(src[e0:e0 + esz], dst[e0:e0 + esz], nch)
        g4 = _edge_stages[k](dist3, edge_feat, W1, b1r, W2, b2r)
        accs.append(_sc_scatters[k](ipk, a2, g4.reshape(2 * esz, DIM), zrows))

    out = _final_stage(*[a.reshape(2, NPAD, DIM)[:, :N_NODES, :]
                         for a in accs], W3, b3r)
    return out
